# baseline (speedup 1.0000x reference)
import functools

import jax
import jax.numpy as jnp
from jax.experimental import pallas as pl
from jax.experimental.pallas import tpu as pltpu

_ROWS_BLK = 2000
_SM_COLS = 1000


def _scores_body(hv_ref, w1_ref, w2_ref, out_ref, va_ref):
    d = hv_ref.shape[1]

    @pl.when(pl.program_id(0) == 0)
    def _():
        va_ref[...] = jnp.dot(w1_ref[0:d, :], w2_ref[...],
                              preferred_element_type=jnp.float32)

    out_ref[...] = jnp.dot(hv_ref[...], va_ref[...],
                           preferred_element_type=jnp.float32)


def _softmax_body(s_ref, dest_ref, probs_ref, logp_ref, *, n_valid):
    s = s_ref[...]
    r = jax.lax.broadcasted_iota(jnp.int32, s.shape, 0)
    c = jax.lax.broadcasted_iota(jnp.int32, s.shape, 1)
    flat = r * s.shape[1] + c
    valid = flat < n_valid
    sm = jnp.where(valid, s, jnp.float32(-1e30))
    m = jnp.max(sm)
    e = jnp.where(valid, jnp.exp(sm - m), jnp.float32(0.0))
    tot = jnp.sum(e)
    probs_ref[...] = e / tot
    sd = jnp.sum(jnp.where(flat == dest_ref[0], sm, jnp.float32(0.0)))
    logp_ref[...] = jnp.reshape(sd - m - jnp.log(tot), (1, 1))


def kernel(hv, W1, b1, W2, b2, bond_type, dest):
    n, d = hv.shape
    d_in = W1.shape[0]
    del b1, b2, bond_type

    scores = pl.pallas_call(
        _scores_body,
        grid=(n // _ROWS_BLK,),
        in_specs=[
            pl.BlockSpec((_ROWS_BLK, d), lambda i: (i, 0)),
            pl.BlockSpec((d_in, d_in), lambda i: (0, 0)),
            pl.BlockSpec((d_in, 1), lambda i: (0, 0)),
        ],
        out_specs=pl.BlockSpec((_ROWS_BLK, 1), lambda i: (i, 0)),
        out_shape=jax.ShapeDtypeStruct((n, 1), jnp.float32),
        scratch_shapes=[pltpu.VMEM((d, 1), jnp.float32)],
    )(hv, W1, W2)

    rows = n // _SM_COLS
    s2d = scores.reshape(rows, _SM_COLS)
    dest_arr = jnp.asarray(dest, jnp.int32).reshape(1)

    probs2d, logp = pl.pallas_call(
        functools.partial(_softmax_body, n_valid=n - 1),
        in_specs=[
            pl.BlockSpec((rows, _SM_COLS), lambda: (0, 0)),
            pl.BlockSpec(memory_space=pltpu.SMEM),
        ],
        out_specs=[
            pl.BlockSpec((rows, _SM_COLS), lambda: (0, 0)),
            pl.BlockSpec((1, 1), lambda: (0, 0)),
        ],
        out_shape=[
            jax.ShapeDtypeStruct((rows, _SM_COLS), jnp.float32),
            jax.ShapeDtypeStruct((1, 1), jnp.float32),
        ],
    )(s2d, dest_arr)

    probs = probs2d.reshape(1, n)[:, : n - 1]
    return probs, logp


# speedup vs baseline: 1.7794x; 1.7794x over previous
import functools

import jax
import jax.numpy as jnp
from jax.experimental import pallas as pl
from jax.experimental.pallas import tpu as pltpu

_ROWS_BLK = 2000


def _fused_body(hv_ref, w1_ref, w2r_ref, dest_ref, probs_ref, logp_ref,
                va_ref, s_ref, *, n_rows, n_valid):
    i = pl.program_id(0)
    nsteps = pl.num_programs(0)
    d = hv_ref.shape[1]

    @pl.when(i == 0)
    def _():
        va_ref[...] = jax.lax.dot_general(
            w2r_ref[...], w1_ref[0:d, :], (((1,), (1,)), ((), ())),
            preferred_element_type=jnp.float32)

    s_blk = jax.lax.dot_general(
        va_ref[...], hv_ref[...], (((1,), (1,)), ((), ())),
        preferred_element_type=jnp.float32)
    s_ref[pl.ds(i, 1), :] = s_blk

    @pl.when(i == nsteps - 1)
    def _():
        s = s_ref[...]
        r = jax.lax.broadcasted_iota(jnp.int32, s.shape, 0)
        c = jax.lax.broadcasted_iota(jnp.int32, s.shape, 1)
        flat = r * s.shape[1] + c
        valid = flat < n_valid
        sm = jnp.where(valid, s, jnp.float32(-1e30))
        m = jnp.max(sm)
        e = jnp.where(valid, jnp.exp(sm - m), jnp.float32(0.0))
        tot = jnp.sum(e)
        probs_ref[...] = e / tot
        sd = jnp.sum(jnp.where(flat == dest_ref[0], sm, jnp.float32(0.0)))
        logp_ref[...] = jnp.reshape(sd - m - jnp.log(tot), (1, 1))


def kernel(hv, W1, b1, W2, b2, bond_type, dest):
    n, d = hv.shape
    d_in = W1.shape[0]
    del b1, b2, bond_type

    nsteps = n // _ROWS_BLK
    w2r = W2.reshape(1, d_in)
    dest_arr = jnp.asarray(dest, jnp.int32).reshape(1)

    probs2d, logp = pl.pallas_call(
        functools.partial(_fused_body, n_rows=n, n_valid=n - 1),
        grid=(nsteps,),
        in_specs=[
            pl.BlockSpec((_ROWS_BLK, d), lambda i: (i, 0)),
            pl.BlockSpec((d_in, d_in), lambda i: (0, 0)),
            pl.BlockSpec((1, d_in), lambda i: (0, 0)),
            pl.BlockSpec(memory_space=pltpu.SMEM),
        ],
        out_specs=[
            pl.BlockSpec((nsteps, _ROWS_BLK), lambda i: (0, 0)),
            pl.BlockSpec((1, 1), lambda i: (0, 0)),
        ],
        out_shape=[
            jax.ShapeDtypeStruct((nsteps, _ROWS_BLK), jnp.float32),
            jax.ShapeDtypeStruct((1, 1), jnp.float32),
        ],
        scratch_shapes=[
            pltpu.VMEM((1, d), jnp.float32),
            pltpu.VMEM((nsteps, _ROWS_BLK), jnp.float32),
        ],
    )(hv, W1, w2r, dest_arr)

    probs = probs2d.reshape(1, n)[:, : n - 1]
    return probs, logp


# speedup vs baseline: 2.5418x; 1.4285x over previous
import functools

import jax
import jax.numpy as jnp
from jax.experimental import pallas as pl
from jax.experimental.pallas import tpu as pltpu

_ROWS_BLK = 4000


def _fused_body(hv_ref, w1_ref, w2r_ref, dest_ref, probs_ref, logp_ref,
                va_ref, s_ref, *, n_rows, n_valid):
    i = pl.program_id(0)
    nsteps = pl.num_programs(0)
    d = hv_ref.shape[1]

    @pl.when(i == 0)
    def _():
        va_ref[...] = jax.lax.dot_general(
            w2r_ref[...], w1_ref[0:d, :], (((1,), (1,)), ((), ())),
            preferred_element_type=jnp.float32)

    s_blk = jax.lax.dot_general(
        va_ref[...], hv_ref[...], (((1,), (1,)), ((), ())),
        preferred_element_type=jnp.float32)
    s_ref[pl.ds(i, 1), :] = s_blk

    @pl.when(i == nsteps - 1)
    def _():
        s = s_ref[...]
        r = jax.lax.broadcasted_iota(jnp.int32, s.shape, 0)
        c = jax.lax.broadcasted_iota(jnp.int32, s.shape, 1)
        flat = r * s.shape[1] + c
        valid = flat < n_valid
        sm = jnp.where(valid, s, jnp.float32(-1e30))
        m = jnp.max(sm)
        e = jnp.where(valid, jnp.exp(sm - m), jnp.float32(0.0))
        tot = jnp.sum(e)
        probs_ref[...] = e / tot
        sd = jnp.sum(jnp.where(flat == dest_ref[0], sm, jnp.float32(0.0)))
        logp_ref[...] = jnp.reshape(sd - m - jnp.log(tot), (1, 1))


def kernel(hv, W1, b1, W2, b2, bond_type, dest):
    n, d = hv.shape
    d_in = W1.shape[0]
    del b1, b2, bond_type

    nsteps = n // _ROWS_BLK
    w2r = W2.reshape(1, d_in)
    dest_arr = jnp.asarray(dest, jnp.int32).reshape(1)

    probs2d, logp = pl.pallas_call(
        functools.partial(_fused_body, n_rows=n, n_valid=n - 1),
        grid=(nsteps,),
        in_specs=[
            pl.BlockSpec((_ROWS_BLK, d), lambda i: (i, 0)),
            pl.BlockSpec((d_in, d_in), lambda i: (0, 0)),
            pl.BlockSpec((1, d_in), lambda i: (0, 0)),
            pl.BlockSpec(memory_space=pltpu.SMEM),
        ],
        out_specs=[
            pl.BlockSpec((nsteps, _ROWS_BLK), lambda i: (0, 0)),
            pl.BlockSpec((1, 1), lambda i: (0, 0)),
        ],
        out_shape=[
            jax.ShapeDtypeStruct((nsteps, _ROWS_BLK), jnp.float32),
            jax.ShapeDtypeStruct((1, 1), jnp.float32),
        ],
        scratch_shapes=[
            pltpu.VMEM((1, d), jnp.float32),
            pltpu.VMEM((nsteps, _ROWS_BLK), jnp.float32),
        ],
    )(hv, W1, w2r, dest_arr)

    probs = probs2d.reshape(1, n)[:, : n - 1]
    return probs, logp


# speedup vs baseline: 3.1900x; 1.2550x over previous
import functools

import jax
import jax.numpy as jnp
from jax.experimental import pallas as pl
from jax.experimental.pallas import tpu as pltpu

_ROWS_BLK = 10000


def _scores_body(hv_ref, w1_ref, w2r_ref, out_ref, va_ref):
    d = hv_ref.shape[1]

    @pl.when(pl.program_id(0) == 0)
    def _():
        va_ref[...] = jax.lax.dot_general(
            w2r_ref[...], w1_ref[0:d, :], (((1,), (1,)), ((), ())),
            preferred_element_type=jnp.float32)

    s_blk = jax.lax.dot_general(
        va_ref[...], hv_ref[...], (((1,), (1,)), ((), ())),
        preferred_element_type=jnp.float32)
    out_ref[...] = s_blk.reshape(out_ref.shape)


def _softmax_body(s_ref, dest_ref, probs_ref, logp_ref, *, n_valid):
    s = s_ref[...].reshape(s_ref.shape[0], s_ref.shape[2])
    r = jax.lax.broadcasted_iota(jnp.int32, s.shape, 0)
    c = jax.lax.broadcasted_iota(jnp.int32, s.shape, 1)
    flat = r * s.shape[1] + c
    valid = flat < n_valid
    sm = jnp.where(valid, s, jnp.float32(-1e30))
    m = jnp.max(sm)
    e = jnp.where(valid, jnp.exp(sm - m), jnp.float32(0.0))
    tot = jnp.sum(e)
    probs_ref[...] = e / tot
    sd = jnp.sum(jnp.where(flat == dest_ref[0], sm, jnp.float32(0.0)))
    logp_ref[...] = jnp.reshape(sd - m - jnp.log(tot), (1, 1))


def kernel(hv, W1, b1, W2, b2, bond_type, dest):
    n, d = hv.shape
    d_in = W1.shape[0]
    del b1, b2, bond_type

    nsteps = n // _ROWS_BLK
    w2r = W2.reshape(1, d_in)
    dest_arr = jnp.asarray(dest, jnp.int32).reshape(1)

    scores = pl.pallas_call(
        _scores_body,
        grid=(nsteps,),
        in_specs=[
            pl.BlockSpec((_ROWS_BLK, d), lambda i: (i, 0)),
            pl.BlockSpec((d_in, d_in), lambda i: (0, 0)),
            pl.BlockSpec((1, d_in), lambda i: (0, 0)),
        ],
        out_specs=pl.BlockSpec((1, 1, _ROWS_BLK), lambda i: (i, 0, 0)),
        out_shape=jax.ShapeDtypeStruct((nsteps, 1, _ROWS_BLK), jnp.float32),
        scratch_shapes=[pltpu.VMEM((1, d), jnp.float32)],
    )(hv, W1, w2r)

    probs2d, logp = pl.pallas_call(
        functools.partial(_softmax_body, n_valid=n - 1),
        in_specs=[
            pl.BlockSpec((nsteps, 1, _ROWS_BLK), lambda: (0, 0, 0)),
            pl.BlockSpec(memory_space=pltpu.SMEM),
        ],
        out_specs=[
            pl.BlockSpec((nsteps, _ROWS_BLK), lambda: (0, 0)),
            pl.BlockSpec((1, 1), lambda: (0, 0)),
        ],
        out_shape=[
            jax.ShapeDtypeStruct((nsteps, _ROWS_BLK), jnp.float32),
            jax.ShapeDtypeStruct((1, 1), jnp.float32),
        ],
    )(scores, dest_arr)

    probs = probs2d.reshape(1, n)[:, : n - 1]
    return probs, logp
